# baseline (speedup 1.0000x reference)
import functools
import jax
import jax.numpy as jnp
from jax.experimental import pallas as pl
from jax.experimental.pallas import tpu as pltpu

ALPHA = 16.0


def _fused_kernel(x_ref, wcat_ref, ball_ref, o_ref, *, n_exp, rank, d_out):
    er = n_exp * rank
    xt = x_ref[...].astype(wcat_ref.dtype)
    acc = jnp.dot(xt, wcat_ref[...], preferred_element_type=jnp.float32)
    base = acc[:, :d_out]
    za = acc[:, d_out:d_out + er]
    l64 = acc[:, d_out + er:]

    ec = (jax.lax.broadcasted_iota(jnp.int32, l64.shape, 1) // rank).astype(jnp.float32)
    big = jnp.float32(n_exp)
    m1 = jnp.max(l64, axis=1, keepdims=True)
    a1 = jnp.min(jnp.where(l64 == m1, ec, big), axis=1, keepdims=True)
    l2 = jnp.where(ec == a1, -jnp.inf, l64)
    m2 = jnp.max(l2, axis=1, keepdims=True)
    a2 = jnp.min(jnp.where(l2 == m2, ec, big), axis=1, keepdims=True)
    w1 = 1.0 / (1.0 + jnp.exp(m2 - m1))
    w2 = 1.0 - w1

    scale = jnp.where(ec == a1, w1, 0.0) + jnp.where(ec == a2, w2, 0.0)
    zb = (za * (scale * ALPHA)).astype(ball_ref.dtype)
    o_ref[...] = base + jnp.dot(zb, ball_ref[...], preferred_element_type=jnp.float32)


def kernel(x, W, gate_W, A, Bm):
    Bb, S, H = x.shape
    OUT = W.shape[0]
    E, R, _ = A.shape
    ER = E * R
    N = Bb * S
    xf = x.reshape(N, H)
    gWt_rep = jnp.repeat(gate_W.T, R, axis=1)
    Wcat = jnp.concatenate(
        [W.T, A.reshape(ER, H).T, gWt_rep], axis=1
    ).astype(jnp.bfloat16)
    Ball = jnp.transpose(Bm, (0, 2, 1)).reshape(ER, OUT).astype(jnp.bfloat16)

    T = 512 if N % 512 == 0 else N
    body = functools.partial(_fused_kernel, n_exp=E, rank=R, d_out=OUT)
    out = pl.pallas_call(
        body,
        grid=(N // T,),
        in_specs=[
            pl.BlockSpec((T, H), lambda i: (i, 0)),
            pl.BlockSpec((H, OUT + 2 * ER), lambda i: (0, 0)),
            pl.BlockSpec((ER, OUT), lambda i: (0, 0)),
        ],
        out_specs=pl.BlockSpec((T, OUT), lambda i: (i, 0)),
        out_shape=jax.ShapeDtypeStruct((N, OUT), jnp.float32),
        compiler_params=pltpu.CompilerParams(dimension_semantics=("parallel",)),
    )(xf, Wcat, Ball)
    return out.reshape(Bb, S, OUT)


# speedup vs baseline: 1.0197x; 1.0197x over previous
import functools
import jax
import jax.numpy as jnp
from jax.experimental import pallas as pl
from jax.experimental.pallas import tpu as pltpu

ALPHA = 16.0


def _fused_kernel(x_ref, wt_ref, acat_ref, ball_ref, o_ref, *, n_exp, rank, d_out):
    er = n_exp * rank
    xt = x_ref[...].astype(wt_ref.dtype)
    acc = jnp.dot(xt, acat_ref[...], preferred_element_type=jnp.float32)
    za = acc[:, :er]
    l64 = acc[:, er:]

    ec = (jax.lax.broadcasted_iota(jnp.int32, l64.shape, 1) // rank).astype(jnp.float32)
    big = jnp.float32(n_exp)
    m1 = jnp.max(l64, axis=1, keepdims=True)
    a1 = jnp.min(jnp.where(l64 == m1, ec, big), axis=1, keepdims=True)
    l2 = jnp.where(ec == a1, -jnp.inf, l64)
    m2 = jnp.max(l2, axis=1, keepdims=True)
    a2 = jnp.min(jnp.where(l2 == m2, ec, big), axis=1, keepdims=True)
    w1 = 1.0 / (1.0 + jnp.exp(m2 - m1))
    w2 = 1.0 - w1

    scale = jnp.where(ec == a1, w1, 0.0) + jnp.where(ec == a2, w2, 0.0)
    zb = (za * (scale * ALPHA)).astype(ball_ref.dtype)
    base = jnp.dot(xt, wt_ref[...], preferred_element_type=jnp.float32)
    o_ref[...] = base + jnp.dot(zb, ball_ref[...], preferred_element_type=jnp.float32)


def kernel(x, W, gate_W, A, Bm):
    Bb, S, H = x.shape
    OUT = W.shape[0]
    E, R, _ = A.shape
    ER = E * R
    N = Bb * S
    xf = x.reshape(N, H)
    gWt_rep = jnp.repeat(gate_W.T, R, axis=1)
    Wt = W.T.astype(jnp.bfloat16)
    Acat = jnp.concatenate(
        [A.reshape(ER, H).T, gWt_rep], axis=1
    ).astype(jnp.bfloat16)
    Ball = jnp.transpose(Bm, (0, 2, 1)).reshape(ER, OUT).astype(jnp.bfloat16)

    T = 512 if N % 512 == 0 else N
    body = functools.partial(_fused_kernel, n_exp=E, rank=R, d_out=OUT)
    out = pl.pallas_call(
        body,
        grid=(N // T,),
        in_specs=[
            pl.BlockSpec((T, H), lambda i: (i, 0)),
            pl.BlockSpec((H, OUT), lambda i: (0, 0)),
            pl.BlockSpec((H, 2 * ER), lambda i: (0, 0)),
            pl.BlockSpec((ER, OUT), lambda i: (0, 0)),
        ],
        out_specs=pl.BlockSpec((T, OUT), lambda i: (i, 0)),
        out_shape=jax.ShapeDtypeStruct((N, OUT), jnp.float32),
        compiler_params=pltpu.CompilerParams(dimension_semantics=("parallel",)),
    )(xf, Wt, Acat, Ball)
    return out.reshape(Bb, S, OUT)


# speedup vs baseline: 1.3762x; 1.3496x over previous
import functools
import jax
import jax.numpy as jnp
from jax.experimental import pallas as pl
from jax.experimental.pallas import tpu as pltpu

ALPHA = 16.0


def _fused_kernel(x_ref, wt_ref, gwt_ref, at_ref, ball_ref, o_ref, *, n_exp, rank):
    xt = x_ref[...].astype(wt_ref.dtype)
    base = jnp.dot(xt, wt_ref[...], preferred_element_type=jnp.float32)
    logits = jnp.dot(xt, gwt_ref[...], preferred_element_type=jnp.float32)

    eio = jax.lax.broadcasted_iota(jnp.int32, logits.shape, 1)
    m1 = jnp.max(logits, axis=1, keepdims=True)
    a1 = jnp.min(jnp.where(logits == m1, eio, n_exp), axis=1, keepdims=True)
    l2 = jnp.where(eio == a1, -jnp.inf, logits)
    m2 = jnp.max(l2, axis=1, keepdims=True)
    a2 = jnp.min(jnp.where(l2 == m2, eio, n_exp), axis=1, keepdims=True)
    w1 = 1.0 / (1.0 + jnp.exp(m2 - m1))
    w2 = 1.0 - w1

    za = jnp.dot(xt, at_ref[...], preferred_element_type=jnp.float32)
    col_exp = jax.lax.broadcasted_iota(jnp.int32, za.shape, 1) // rank
    scale = jnp.where(col_exp == a1, w1, 0.0) + jnp.where(col_exp == a2, w2, 0.0)
    zb = (za * (scale * ALPHA)).astype(ball_ref.dtype)
    o_ref[...] = base + jnp.dot(zb, ball_ref[...], preferred_element_type=jnp.float32)


def kernel(x, W, gate_W, A, Bm):
    Bb, S, H = x.shape
    OUT = W.shape[0]
    E, R, _ = A.shape
    ER = E * R
    N = Bb * S
    xf = x.reshape(N, H)
    Wt = W.T.astype(jnp.bfloat16)
    gWt = gate_W.T.astype(jnp.bfloat16)
    At = A.reshape(ER, H).T.astype(jnp.bfloat16)
    Ball = jnp.transpose(Bm, (0, 2, 1)).reshape(ER, OUT).astype(jnp.bfloat16)

    T = 1024 if N % 1024 == 0 else N
    body = functools.partial(_fused_kernel, n_exp=E, rank=R)
    out = pl.pallas_call(
        body,
        grid=(N // T,),
        in_specs=[
            pl.BlockSpec((T, H), lambda i: (i, 0)),
            pl.BlockSpec((H, OUT), lambda i: (0, 0)),
            pl.BlockSpec((H, E), lambda i: (0, 0)),
            pl.BlockSpec((H, ER), lambda i: (0, 0)),
            pl.BlockSpec((ER, OUT), lambda i: (0, 0)),
        ],
        out_specs=pl.BlockSpec((T, OUT), lambda i: (i, 0)),
        out_shape=jax.ShapeDtypeStruct((N, OUT), jnp.float32),
        compiler_params=pltpu.CompilerParams(dimension_semantics=("parallel",)),
    )(xf, Wt, gWt, At, Ball)
    return out.reshape(Bb, S, OUT)


# speedup vs baseline: 1.3933x; 1.0124x over previous
import functools
import jax
import jax.numpy as jnp
from jax.experimental import pallas as pl
from jax.experimental.pallas import tpu as pltpu

ALPHA = 16.0


def _fused_kernel(x_ref, wt_ref, gwt_ref, at_ref, ball_ref, o_ref, *, n_exp, rank):
    xt = x_ref[...].astype(wt_ref.dtype)
    base = jnp.dot(xt, wt_ref[...], preferred_element_type=jnp.float32)
    logits = jnp.dot(xt, gwt_ref[...], preferred_element_type=jnp.float32)

    eio = jax.lax.broadcasted_iota(jnp.int32, logits.shape, 1)
    m1 = jnp.max(logits, axis=1, keepdims=True)
    a1 = jnp.min(jnp.where(logits == m1, eio, n_exp), axis=1, keepdims=True)
    l2 = jnp.where(eio == a1, -jnp.inf, logits)
    m2 = jnp.max(l2, axis=1, keepdims=True)
    a2 = jnp.min(jnp.where(l2 == m2, eio, n_exp), axis=1, keepdims=True)
    w1 = 1.0 / (1.0 + jnp.exp(m2 - m1))
    w2 = 1.0 - w1

    za = jnp.dot(xt, at_ref[...], preferred_element_type=jnp.float32)
    col_exp = jax.lax.broadcasted_iota(jnp.int32, za.shape, 1) // rank
    scale = jnp.where(col_exp == a1, w1, 0.0) + jnp.where(col_exp == a2, w2, 0.0)
    zb = (za * (scale * ALPHA)).astype(ball_ref.dtype)
    o_ref[...] = base + jnp.dot(zb, ball_ref[...], preferred_element_type=jnp.float32)


def kernel(x, W, gate_W, A, Bm):
    Bb, S, H = x.shape
    OUT = W.shape[0]
    E, R, _ = A.shape
    ER = E * R
    N = Bb * S
    xf = x.reshape(N, H)
    Wt = W.T.astype(jnp.bfloat16)
    gWt = gate_W.T.astype(jnp.bfloat16)
    At = A.reshape(ER, H).T.astype(jnp.bfloat16)
    Ball = jnp.transpose(Bm, (0, 2, 1)).reshape(ER, OUT).astype(jnp.bfloat16)

    T = 2048 if N % 2048 == 0 else N
    body = functools.partial(_fused_kernel, n_exp=E, rank=R)
    out = pl.pallas_call(
        body,
        grid=(N // T,),
        in_specs=[
            pl.BlockSpec((T, H), lambda i: (i, 0)),
            pl.BlockSpec((H, OUT), lambda i: (0, 0)),
            pl.BlockSpec((H, E), lambda i: (0, 0)),
            pl.BlockSpec((H, ER), lambda i: (0, 0)),
            pl.BlockSpec((ER, OUT), lambda i: (0, 0)),
        ],
        out_specs=pl.BlockSpec((T, OUT), lambda i: (i, 0)),
        out_shape=jax.ShapeDtypeStruct((N, OUT), jnp.float32),
        compiler_params=pltpu.CompilerParams(dimension_semantics=("parallel",)),
    )(xf, Wt, gWt, At, Ball)
    return out.reshape(Bb, S, OUT)


# speedup vs baseline: 1.4198x; 1.0190x over previous
import functools
import jax
import jax.numpy as jnp
from jax.experimental import pallas as pl
from jax.experimental.pallas import tpu as pltpu

ALPHA = 16.0


def _fused_kernel(x_ref, wt_ref, gwt_ref, at_ref, ball_ref, o_ref, *, n_exp, rank):
    xt = x_ref[...].astype(wt_ref.dtype)
    base = jnp.dot(xt, wt_ref[...], preferred_element_type=jnp.float32)
    logits = jnp.dot(xt, gwt_ref[...], preferred_element_type=jnp.float32)

    eio = jax.lax.broadcasted_iota(jnp.int32, logits.shape, 1)
    m1 = jnp.max(logits, axis=1, keepdims=True)
    a1 = jnp.min(jnp.where(logits == m1, eio, n_exp), axis=1, keepdims=True)
    l2 = jnp.where(eio == a1, -jnp.inf, logits)
    m2 = jnp.max(l2, axis=1, keepdims=True)
    a2 = jnp.min(jnp.where(l2 == m2, eio, n_exp), axis=1, keepdims=True)
    w1 = 1.0 / (1.0 + jnp.exp(m2 - m1))
    w2 = 1.0 - w1

    za = jnp.dot(xt, at_ref[...], preferred_element_type=jnp.float32)
    col_exp = jax.lax.broadcasted_iota(jnp.int32, za.shape, 1) // rank
    scale = jnp.where(col_exp == a1, w1, 0.0) + jnp.where(col_exp == a2, w2, 0.0)
    zb = (za * (scale * ALPHA)).astype(ball_ref.dtype)
    o_ref[...] = base + jnp.dot(zb, ball_ref[...], preferred_element_type=jnp.float32)


def kernel(x, W, gate_W, A, Bm):
    Bb, S, H = x.shape
    OUT = W.shape[0]
    E, R, _ = A.shape
    ER = E * R
    N = Bb * S
    xf = x.reshape(N, H)
    Wt = W.T
    gWt = gate_W.T
    At = A.reshape(ER, H).T
    Ball = jnp.transpose(Bm, (0, 2, 1)).reshape(ER, OUT)

    T = 2048 if N % 2048 == 0 else N
    body = functools.partial(_fused_kernel, n_exp=E, rank=R)
    out = pl.pallas_call(
        body,
        grid=(N // T,),
        in_specs=[
            pl.BlockSpec((T, H), lambda i: (i, 0)),
            pl.BlockSpec((H, OUT), lambda i: (0, 0)),
            pl.BlockSpec((H, E), lambda i: (0, 0)),
            pl.BlockSpec((H, ER), lambda i: (0, 0)),
            pl.BlockSpec((ER, OUT), lambda i: (0, 0)),
        ],
        out_specs=pl.BlockSpec((T, OUT), lambda i: (i, 0)),
        out_shape=jax.ShapeDtypeStruct((N, OUT), jnp.float32),
        compiler_params=pltpu.CompilerParams(dimension_semantics=("parallel",)),
    )(xf, Wt, gWt, At, Ball)
    return out.reshape(Bb, S, OUT)


# speedup vs baseline: 1.5488x; 1.0909x over previous
import functools
import jax
import jax.numpy as jnp
from jax.experimental import pallas as pl
from jax.experimental.pallas import tpu as pltpu

ALPHA = 16.0


def _fused_kernel(x_ref, wt_ref, acat_ref, ball_ref, o_ref, *, n_exp, rank):
    er = n_exp * rank
    xt = x_ref[...]
    acc = jnp.dot(xt, acat_ref[...], preferred_element_type=jnp.float32)

    lio = jax.lax.broadcasted_iota(jnp.int32, acc.shape, 1)
    lmask = jnp.where(lio >= er, acc, -jnp.inf)
    m1 = jnp.max(lmask, axis=1, keepdims=True)
    a1 = jnp.min(jnp.where(lmask == m1, lio, er + n_exp), axis=1, keepdims=True)
    l2 = jnp.where(lio == a1, -jnp.inf, lmask)
    m2 = jnp.max(l2, axis=1, keepdims=True)
    a2 = jnp.min(jnp.where(l2 == m2, lio, er + n_exp), axis=1, keepdims=True)
    w1 = 1.0 / (1.0 + jnp.exp(m2 - m1))
    w2 = 1.0 - w1

    col_exp = lio // rank
    scale = jnp.where(col_exp == a1 - er, w1, 0.0) + jnp.where(col_exp == a2 - er, w2, 0.0)
    zb = (acc * (scale * ALPHA))[:, :er]
    base = jnp.dot(xt, wt_ref[...], preferred_element_type=jnp.float32)
    o_ref[...] = base + jnp.dot(zb, ball_ref[...], preferred_element_type=jnp.float32)


def kernel(x, W, gate_W, A, Bm):
    Bb, S, H = x.shape
    OUT = W.shape[0]
    E, R, _ = A.shape
    ER = E * R
    N = Bb * S
    xf = x.reshape(N, H)
    Wt = W.T
    Acat = jnp.concatenate([A.reshape(ER, H).T, gate_W.T], axis=1)
    Ball = jnp.transpose(Bm, (0, 2, 1)).reshape(ER, OUT)

    T = 1024 if N % 1024 == 0 else N
    body = functools.partial(_fused_kernel, n_exp=E, rank=R)
    out = pl.pallas_call(
        body,
        grid=(N // T,),
        in_specs=[
            pl.BlockSpec((T, H), lambda i: (i, 0)),
            pl.BlockSpec((H, OUT), lambda i: (0, 0)),
            pl.BlockSpec((H, ER + E), lambda i: (0, 0)),
            pl.BlockSpec((ER, OUT), lambda i: (0, 0)),
        ],
        out_specs=pl.BlockSpec((T, OUT), lambda i: (i, 0)),
        out_shape=jax.ShapeDtypeStruct((N, OUT), jnp.float32),
        compiler_params=pltpu.CompilerParams(dimension_semantics=("parallel",)),
    )(xf, Wt, Acat, Ball)
    return out.reshape(Bb, S, OUT)


# speedup vs baseline: 1.6257x; 1.0497x over previous
import functools
import jax
import jax.numpy as jnp
from jax.experimental import pallas as pl
from jax.experimental.pallas import tpu as pltpu

ALPHA = 16.0


def _fused_kernel(x_ref, wt_ref, acat_ref, ball_ref, o_ref, *, n_exp, rank):
    er = n_exp * rank
    xt = x_ref[...]
    acc = jnp.dot(xt, acat_ref[...], preferred_element_type=jnp.float32)

    lio = jax.lax.broadcasted_iota(jnp.int32, acc.shape, 1)
    lmask = jnp.where(lio >= er, acc, -jnp.inf)
    m1 = jnp.max(lmask, axis=1, keepdims=True)
    a1 = jnp.min(jnp.where(lmask == m1, lio, er + n_exp), axis=1, keepdims=True)
    l2 = jnp.where(lio == a1, -jnp.inf, lmask)
    m2 = jnp.max(l2, axis=1, keepdims=True)
    a2 = jnp.min(jnp.where(l2 == m2, lio, er + n_exp), axis=1, keepdims=True)
    w1 = 1.0 / (1.0 + jnp.exp(m2 - m1))
    w2 = 1.0 - w1

    col_exp = lio // rank
    scale = jnp.where(col_exp == a1 - er, w1, 0.0) + jnp.where(col_exp == a2 - er, w2, 0.0)
    zb = (acc * (scale * ALPHA))[:, :er]
    base = jnp.dot(xt, wt_ref[...], preferred_element_type=jnp.float32)
    o_ref[...] = base + jnp.dot(zb, ball_ref[...], preferred_element_type=jnp.float32)


def kernel(x, W, gate_W, A, Bm):
    Bb, S, H = x.shape
    OUT = W.shape[0]
    E, R, _ = A.shape
    ER = E * R
    N = Bb * S
    xf = x.reshape(N, H)
    Wt = W.T
    Acat = jnp.concatenate([A.reshape(ER, H).T, gate_W.T], axis=1)
    Ball = jnp.transpose(Bm, (0, 2, 1)).reshape(ER, OUT)

    T = 2048 if N % 2048 == 0 else N
    body = functools.partial(_fused_kernel, n_exp=E, rank=R)
    out = pl.pallas_call(
        body,
        grid=(N // T,),
        in_specs=[
            pl.BlockSpec((T, H), lambda i: (i, 0)),
            pl.BlockSpec((H, OUT), lambda i: (0, 0)),
            pl.BlockSpec((H, ER + E), lambda i: (0, 0)),
            pl.BlockSpec((ER, OUT), lambda i: (0, 0)),
        ],
        out_specs=pl.BlockSpec((T, OUT), lambda i: (i, 0)),
        out_shape=jax.ShapeDtypeStruct((N, OUT), jnp.float32),
        compiler_params=pltpu.CompilerParams(dimension_semantics=("parallel",)),
    )(xf, Wt, Acat, Ball)
    return out.reshape(Bb, S, OUT)
